# baseline (speedup 1.0000x reference)
import functools

import jax
import jax.numpy as jnp
import numpy as np
from jax import lax
from jax.experimental import pallas as pl
from jax.experimental.pallas import tpu as pltpu
from jax.experimental.pallas import tpu_sc as plsc

VOCAB = 100000
EMBED = 128
NC, NS = 2, 16
NW = NC * NS

_consts = {}


def _keep_u8():
    if "keep" not in _consts:
        with jax.ensure_compile_time_eval():
            keep = jax.random.bernoulli(
                jax.random.key(42), 0.5, (VOCAB, EMBED)
            )
            _consts["keep"] = np.asarray(keep).astype(np.uint8)
    return _consts["keep"]


def _mask_body(w_ref, k_ref, o_ref):
    o_ref[...] = jnp.where(k_ref[...] != 0, w_ref[...] * 2.0, 0.0)


def _masked_table(weight_raw, keep_u8):
    br = 4000
    grid = VOCAB // br
    return pl.pallas_call(
        _mask_body,
        grid=(grid,),
        in_specs=[
            pl.BlockSpec((br, EMBED), lambda i: (i, 0)),
            pl.BlockSpec((br, EMBED), lambda i: (i, 0)),
        ],
        out_specs=pl.BlockSpec((br, EMBED), lambda i: (i, 0)),
        out_shape=jax.ShapeDtypeStruct((VOCAB, EMBED), jnp.float32),
    )(weight_raw, keep_u8)


def _gather(w, idx_t, batch, hist):
    b_per_tile = batch // NW
    sub = 128
    n_sub = b_per_tile // sub
    mesh = plsc.VectorSubcoreMesh(
        core_axis_name="c", subcore_axis_name="s", num_cores=NC, num_subcores=NS
    )

    n_iter = hist * n_sub

    nbuf = 5

    @functools.partial(
        pl.kernel,
        mesh=mesh,
        out_type=jax.ShapeDtypeStruct((hist, batch, EMBED), jnp.float32),
        scratch_types=(
            [pltpu.VMEM((hist, b_per_tile), jnp.int32)]
            + [pltpu.VMEM((sub, EMBED), jnp.float32)] * nbuf
            + [pltpu.SemaphoreType.DMA] * (2 * nbuf)
        ),
        compiler_params=pltpu.CompilerParams(use_tc_tiling_on_sc=True),
    )
    def k(w_hbm, idx_hbm, out_hbm, idx_v, *rest):
        bufs = rest[:nbuf]
        gsems = rest[nbuf:2 * nbuf]
        wsems = rest[2 * nbuf:]
        wid = lax.axis_index("s") * NC + lax.axis_index("c")
        b0 = wid * b_per_tile
        pltpu.sync_copy(idx_hbm.at[:, pl.ds(b0, b_per_tile)], idx_v)

        def src(j):
            h = j >> 2
            s = j & 3
            return w_hbm.at[idx_v.at[h, pl.ds(s * sub, sub)]]

        def dst(j):
            h = j >> 2
            s = j & 3
            return out_hbm.at[h, pl.ds(b0 + s * sub, sub)]

        pltpu.async_copy(src(0), bufs[0], gsems[0])
        pltpu.async_copy(src(1), bufs[1], gsems[1])
        pltpu.async_copy(src(2), bufs[2], gsems[2])

        def body(g, carry):
            for t in range(nbuf):
                j = g * nbuf + t
                tb = (t + 3) % nbuf
                pltpu.make_async_copy(src(j), bufs[t], gsems[t]).wait()
                pltpu.async_copy(bufs[t], dst(j), wsems[t])

                @pl.when(j >= 2)
                def _():
                    pltpu.make_async_copy(bufs[tb], dst(j - 2), wsems[tb]).wait()

                @pl.when(j + 3 < n_iter)
                def _():
                    pltpu.async_copy(src(j + 3), bufs[tb], gsems[tb])
            return carry

        lax.fori_loop(0, n_iter // nbuf, body, 0)
        pltpu.make_async_copy(bufs[3], dst(n_iter - 2), wsems[3]).wait()
        pltpu.make_async_copy(bufs[4], dst(n_iter - 1), wsems[4]).wait()

    return k(w, idx_t)


def kernel(input, weight_raw):
    keep = jnp.asarray(_keep_u8())
    w = _masked_table(weight_raw, keep)
    b, h = input.shape
    idx_t = input.T.astype(jnp.int32)
    out = _gather(w, idx_t, b, h)
    return out.transpose(1, 0, 2)


# speedup vs baseline: 1.0124x; 1.0124x over previous
import functools

import jax
import jax.numpy as jnp
import numpy as np
from jax import lax
from jax.experimental import pallas as pl
from jax.experimental.pallas import tpu as pltpu
from jax.experimental.pallas import tpu_sc as plsc

VOCAB = 100000
EMBED = 128
NC, NS = 2, 16
NW = NC * NS

_consts = {}


def _keep_u8():
    if "keep" not in _consts:
        with jax.ensure_compile_time_eval():
            keep = jax.random.bernoulli(
                jax.random.key(42), 0.5, (VOCAB, EMBED)
            )
            _consts["keep"] = np.asarray(keep).astype(np.uint8)
    return _consts["keep"]


def _mask_body(w_ref, k_ref, o_ref):
    o_ref[...] = jnp.where(k_ref[...] != 0, w_ref[...] * 2.0, 0.0)


def _masked_table(weight_raw, keep_u8):
    br = 4000
    grid = VOCAB // br
    return pl.pallas_call(
        _mask_body,
        grid=(grid,),
        in_specs=[
            pl.BlockSpec((br, EMBED), lambda i: (i, 0)),
            pl.BlockSpec((br, EMBED), lambda i: (i, 0)),
        ],
        out_specs=pl.BlockSpec((br, EMBED), lambda i: (i, 0)),
        out_shape=jax.ShapeDtypeStruct((VOCAB, EMBED), jnp.float32),
    )(weight_raw, keep_u8)


def _gather(w, idx_t, batch, hist):
    b_per_tile = batch // NW
    sub = 128
    n_sub = b_per_tile // sub
    mesh = plsc.VectorSubcoreMesh(
        core_axis_name="c", subcore_axis_name="s", num_cores=NC, num_subcores=NS
    )

    n_iter = hist * n_sub

    nbuf = 5

    @functools.partial(
        pl.kernel,
        mesh=mesh,
        out_type=jax.ShapeDtypeStruct((hist, batch, EMBED), jnp.float32),
        scratch_types=(
            [pltpu.VMEM((hist, b_per_tile), jnp.int32)]
            + [pltpu.VMEM((sub, EMBED), jnp.float32)] * nbuf
            + [pltpu.SemaphoreType.DMA] * (2 * nbuf)
        ),
        compiler_params=pltpu.CompilerParams(use_tc_tiling_on_sc=True),
    )
    def k(w_hbm, idx_hbm, out_hbm, idx_v, *rest):
        bufs = rest[:nbuf]
        gsems = rest[nbuf:2 * nbuf]
        wsems = rest[2 * nbuf:]
        wid = lax.axis_index("s") * NC + lax.axis_index("c")
        b0 = wid * b_per_tile
        pltpu.sync_copy(idx_hbm.at[:, pl.ds(b0, b_per_tile)], idx_v)

        def src(j):
            h = j >> 2
            s = j & 3
            return w_hbm.at[idx_v.at[h, pl.ds(s * sub, sub)]]

        def dst(j):
            h = j >> 2
            s = j & 3
            return out_hbm.at[h, pl.ds(b0 + s * sub, sub)]

        pltpu.async_copy(src(0), bufs[0], gsems[0])
        pltpu.async_copy(src(1), bufs[1], gsems[1])

        def body(g, carry):
            for t in range(nbuf):
                j = g * nbuf + t
                tb = (t + 2) % nbuf
                pltpu.make_async_copy(src(j), bufs[t], gsems[t]).wait()
                pltpu.async_copy(bufs[t], dst(j), wsems[t])

                @pl.when(j >= 3)
                def _():
                    pltpu.make_async_copy(bufs[tb], dst(j - 3), wsems[tb]).wait()

                @pl.when(j + 2 < n_iter)
                def _():
                    pltpu.async_copy(src(j + 2), bufs[tb], gsems[tb])
            return carry

        lax.fori_loop(0, n_iter // nbuf, body, 0)
        pltpu.make_async_copy(bufs[2], dst(n_iter - 3), wsems[2]).wait()
        pltpu.make_async_copy(bufs[3], dst(n_iter - 2), wsems[3]).wait()
        pltpu.make_async_copy(bufs[4], dst(n_iter - 1), wsems[4]).wait()

    return k(w, idx_t)


def kernel(input, weight_raw):
    keep = jnp.asarray(_keep_u8())
    w = _masked_table(weight_raw, keep)
    b, h = input.shape
    idx_t = input.T.astype(jnp.int32)
    out = _gather(w, idx_t, b, h)
    return out.transpose(1, 0, 2)


# speedup vs baseline: 1.0295x; 1.0169x over previous
import functools

import jax
import jax.numpy as jnp
import numpy as np
from jax import lax
from jax.experimental import pallas as pl
from jax.experimental.pallas import tpu as pltpu
from jax.experimental.pallas import tpu_sc as plsc

VOCAB = 100000
EMBED = 128
NC, NS = 2, 16
NW = NC * NS

_consts = {}


def _keep_u8():
    if "keep" not in _consts:
        with jax.ensure_compile_time_eval():
            keep = jax.random.bernoulli(
                jax.random.key(42), 0.5, (VOCAB, EMBED)
            )
            _consts["keep"] = np.asarray(keep).astype(np.uint8)
    return _consts["keep"]


def _mask_body(w_ref, k_ref, o_ref):
    o_ref[...] = jnp.where(k_ref[...] != 0, w_ref[...] * 2.0, 0.0)


def _masked_table(weight_raw, keep_u8):
    br = 20000
    grid = VOCAB // br
    return pl.pallas_call(
        _mask_body,
        grid=(grid,),
        in_specs=[
            pl.BlockSpec((br, EMBED), lambda i: (i, 0)),
            pl.BlockSpec((br, EMBED), lambda i: (i, 0)),
        ],
        out_specs=pl.BlockSpec((br, EMBED), lambda i: (i, 0)),
        out_shape=jax.ShapeDtypeStruct((VOCAB, EMBED), jnp.float32),
        compiler_params=pltpu.CompilerParams(
            vmem_limit_bytes=100 * 1024 * 1024
        ),
    )(weight_raw, keep_u8)


def _gather(w, idx_t, batch, hist):
    b_per_tile = batch // NW
    sub = 128
    n_sub = b_per_tile // sub
    mesh = plsc.VectorSubcoreMesh(
        core_axis_name="c", subcore_axis_name="s", num_cores=NC, num_subcores=NS
    )

    n_iter = hist * n_sub

    nbuf = 5

    @functools.partial(
        pl.kernel,
        mesh=mesh,
        out_type=jax.ShapeDtypeStruct((hist, batch, EMBED), jnp.float32),
        scratch_types=(
            [pltpu.VMEM((hist, b_per_tile), jnp.int32)]
            + [pltpu.VMEM((sub, EMBED), jnp.float32)] * nbuf
            + [pltpu.SemaphoreType.DMA] * (2 * nbuf)
        ),
        compiler_params=pltpu.CompilerParams(use_tc_tiling_on_sc=True),
    )
    def k(w_hbm, idx_hbm, out_hbm, idx_v, *rest):
        bufs = rest[:nbuf]
        gsems = rest[nbuf:2 * nbuf]
        wsems = rest[2 * nbuf:]
        wid = lax.axis_index("s") * NC + lax.axis_index("c")
        b0 = wid * b_per_tile
        pltpu.sync_copy(idx_hbm.at[:, pl.ds(b0, b_per_tile)], idx_v)

        def src(j):
            h = j >> 2
            s = j & 3
            return w_hbm.at[idx_v.at[h, pl.ds(s * sub, sub)]]

        def dst(j):
            h = j >> 2
            s = j & 3
            return out_hbm.at[h, pl.ds(b0 + s * sub, sub)]

        pltpu.async_copy(src(0), bufs[0], gsems[0])
        pltpu.async_copy(src(1), bufs[1], gsems[1])

        def body(g, carry):
            for t in range(nbuf):
                j = g * nbuf + t
                tb = (t + 2) % nbuf
                pltpu.make_async_copy(src(j), bufs[t], gsems[t]).wait()
                pltpu.async_copy(bufs[t], dst(j), wsems[t])

                @pl.when(j >= 3)
                def _():
                    pltpu.make_async_copy(bufs[tb], dst(j - 3), wsems[tb]).wait()

                @pl.when(j + 2 < n_iter)
                def _():
                    pltpu.async_copy(src(j + 2), bufs[tb], gsems[tb])
            return carry

        lax.fori_loop(0, n_iter // nbuf, body, 0)
        pltpu.make_async_copy(bufs[2], dst(n_iter - 3), wsems[2]).wait()
        pltpu.make_async_copy(bufs[3], dst(n_iter - 2), wsems[3]).wait()
        pltpu.make_async_copy(bufs[4], dst(n_iter - 1), wsems[4]).wait()

    return k(w, idx_t)


def kernel(input, weight_raw):
    keep = jnp.asarray(_keep_u8())
    w = _masked_table(weight_raw, keep)
    b, h = input.shape
    idx_t = input.T.astype(jnp.int32)
    out = _gather(w, idx_t, b, h)
    return out.transpose(1, 0, 2)
